# Initial kernel scaffold; baseline (speedup 1.0000x reference)
import jax
import jax.numpy as jnp
from jax.experimental import pallas as pl


def kernel(x, hyperedge_labels, W1, b1, W2, b2, attention, Wr, br):
    raise NotImplementedError("write your pallas kernel here")


# speedup vs baseline: 24.9563x; 24.9563x over previous
import jax
import jax.numpy as jnp
from jax import lax
from jax.experimental import pallas as pl
from jax.experimental.pallas import tpu as pltpu

H_HEADS = 4
HP = 8
C_IN = 192
C_OUT = 192
E_EDGES = 200
EP = 256
BN = 3584
N_TOT = 224 * 224
NB = N_TOT // BN


def _prep_body(w1r_ref, b1r_ref, att_ref, w2r_ref, vt_ref, gt_ref, cb_ref, m_ref):
    zeros4 = jnp.zeros((H_HEADS, C_IN), dtype=jnp.float32)
    vrows = [jnp.dot(att_ref[h : h + 1, :], w1r_ref[h],
                     preferred_element_type=jnp.float32)
             for h in range(H_HEADS)]
    vt_ref[...] = lax.concatenate(vrows + [zeros4], 0)
    grows = [lax.dot_general(b1r_ref[h : h + 1, :], w2r_ref[h],
                             (((1,), (1,)), ((), ())),
                             preferred_element_type=jnp.float32)
             for h in range(H_HEADS)]
    gt_ref[...] = lax.concatenate(grows + [zeros4], 0)
    cbv = jnp.sum(att_ref[...] * b1r_ref[...], axis=1, keepdims=True)
    cb8 = lax.concatenate([cbv, jnp.zeros((H_HEADS, 1), jnp.float32)], 0)
    cb_ref[...] = jnp.broadcast_to(cb8, (HP, 128))
    for h in range(H_HEADS):
        m_ref[h] = jnp.dot(w2r_ref[h], w1r_ref[h],
                           preferred_element_type=jnp.float32)


def _scatter_body(x_ref, idx_ref, vt_ref, cb_ref, yout_ref, sout_ref,
                  ys_ref, ss_ref):
    j = pl.program_id(1)

    @pl.when(j == 0)
    def _init():
        ys_ref[...] = jnp.zeros_like(ys_ref)
        ss_ref[...] = jnp.zeros_like(ss_ref)

    xb = x_ref[0]
    idxv = idx_ref[0, 0]
    logits = jnp.dot(vt_ref[...], xb, preferred_element_type=jnp.float32)
    logits = logits + cb_ref[:, :1]
    ex = jnp.exp(logits)
    oht = (lax.broadcasted_iota(jnp.int32, (EP, BN), 0) == idxv
           ).astype(jnp.float32)
    ss_ref[...] += lax.dot_general(ex, oht, (((1,), (1,)), ((), ())),
                                   preferred_element_type=jnp.float32)
    for h in range(H_HEADS):
        ys_ref[h] += lax.dot_general(xb * ex[h : h + 1, :], oht,
                                     (((1,), (1,)), ((), ())),
                                     preferred_element_type=jnp.float32)

    @pl.when(j == NB - 1)
    def _flush():
        yout_ref[0] = ys_ref[...]
        sout_ref[0] = ss_ref[...]


def _edge_body(ys_ref, ss_ref, m_ref, gt_ref, b2_ref, et_ref):
    s = ss_ref[0]
    cnt = jnp.maximum(s[H_HEADS : H_HEADS + 1, :], 1.0)
    cinv = 1.0 / cnt
    dn = jnp.sum(s, axis=1, keepdims=True)
    sn = s * (cinv / dn)
    acc = lax.dot_general(gt_ref[...], sn, (((0,), (0,)), ((), ())),
                          preferred_element_type=jnp.float32)
    for h in range(H_HEADS):
        dnh = jnp.sum(s[h : h + 1, :], axis=1, keepdims=True)
        zh = ys_ref[0, h] * (cinv / dnh)
        acc = acc + jnp.dot(m_ref[h], zh, preferred_element_type=jnp.float32)
    et_ref[0] = acc + b2_ref[...]


def _out_body(x_ref, idx_ref, et_ref, wr_ref, br_ref, o_ref):
    idxv = idx_ref[0, 0]
    oht = (lax.broadcasted_iota(jnp.int32, (EP, BN), 0) == idxv
           ).astype(jnp.float32)
    g = jnp.dot(et_ref[0], oht, preferred_element_type=jnp.float32)
    r = jnp.dot(wr_ref[...], x_ref[0], preferred_element_type=jnp.float32)
    v = g + r + br_ref[...]
    o_ref[0] = jnp.where(v > 0, v, jnp.exp(jnp.minimum(v, 0.0)) - 1.0)


def kernel(x, hyperedge_labels, W1, b1, W2, b2, attention, Wr, br):
    B, C, H, W = x.shape
    N = H * W
    x3 = x.reshape(B, C, N)
    idx = hyperedge_labels.astype(jnp.int32).reshape(B, NB, 1, BN)

    w1r = W1.reshape(H_HEADS, C_OUT, C)
    b1r = b1.reshape(H_HEADS, C_OUT)
    att = attention.reshape(H_HEADS, C_OUT)
    w2r = jnp.transpose(W2.reshape(C_OUT, H_HEADS, C_OUT), (1, 0, 2))
    b2c = b2.reshape(C_OUT, 1)
    brc = br.reshape(C_OUT, 1)

    vt, gt, cb, m = pl.pallas_call(
        _prep_body,
        out_shape=[
            jax.ShapeDtypeStruct((HP, C_IN), jnp.float32),
            jax.ShapeDtypeStruct((HP, C_IN), jnp.float32),
            jax.ShapeDtypeStruct((HP, 128), jnp.float32),
            jax.ShapeDtypeStruct((H_HEADS, C_OUT, C_IN), jnp.float32),
        ],
    )(w1r, b1r, att, w2r)

    yacc, sacc = pl.pallas_call(
        _scatter_body,
        grid=(B, NB),
        in_specs=[
            pl.BlockSpec((1, C, BN), lambda b, j: (b, 0, j)),
            pl.BlockSpec((1, 1, 1, BN), lambda b, j: (b, j, 0, 0)),
            pl.BlockSpec((HP, C_IN), lambda b, j: (0, 0)),
            pl.BlockSpec((HP, 128), lambda b, j: (0, 0)),
        ],
        out_specs=[
            pl.BlockSpec((1, H_HEADS, C_IN, EP), lambda b, j: (b, 0, 0, 0)),
            pl.BlockSpec((1, HP, EP), lambda b, j: (b, 0, 0)),
        ],
        out_shape=[
            jax.ShapeDtypeStruct((B, H_HEADS, C_IN, EP), jnp.float32),
            jax.ShapeDtypeStruct((B, HP, EP), jnp.float32),
        ],
        scratch_shapes=[
            pltpu.VMEM((H_HEADS, C_IN, EP), jnp.float32),
            pltpu.VMEM((HP, EP), jnp.float32),
        ],
    )(x3, idx, vt, cb)

    et = pl.pallas_call(
        _edge_body,
        grid=(B,),
        in_specs=[
            pl.BlockSpec((1, H_HEADS, C_IN, EP), lambda b: (b, 0, 0, 0)),
            pl.BlockSpec((1, HP, EP), lambda b: (b, 0, 0)),
            pl.BlockSpec((H_HEADS, C_OUT, C_IN), lambda b: (0, 0, 0)),
            pl.BlockSpec((HP, C_IN), lambda b: (0, 0)),
            pl.BlockSpec((C_OUT, 1), lambda b: (0, 0)),
        ],
        out_specs=pl.BlockSpec((1, C_OUT, EP), lambda b: (b, 0, 0)),
        out_shape=jax.ShapeDtypeStruct((B, C_OUT, EP), jnp.float32),
    )(yacc, sacc, m, gt, b2c)

    out = pl.pallas_call(
        _out_body,
        grid=(B, NB),
        in_specs=[
            pl.BlockSpec((1, C, BN), lambda b, j: (b, 0, j)),
            pl.BlockSpec((1, 1, 1, BN), lambda b, j: (b, j, 0, 0)),
            pl.BlockSpec((1, C_OUT, EP), lambda b, j: (b, 0, 0)),
            pl.BlockSpec((C_OUT, C_IN), lambda b, j: (0, 0)),
            pl.BlockSpec((C_OUT, 1), lambda b, j: (0, 0)),
        ],
        out_specs=pl.BlockSpec((1, C_OUT, BN), lambda b, j: (b, 0, j)),
        out_shape=jax.ShapeDtypeStruct((B, C_OUT, N), jnp.float32),
    )(x3, idx, et, Wr, brc)

    return out.reshape(B, C_OUT, H, W)


# speedup vs baseline: 25.4289x; 1.0189x over previous
import jax
import jax.numpy as jnp
from jax import lax
from jax.experimental import pallas as pl
from jax.experimental.pallas import tpu as pltpu

H_HEADS = 4
HP = 8
C_IN = 192
C_OUT = 192
E_EDGES = 200
EP = 256
BN = 3584
N_TOT = 224 * 224
NB = N_TOT // BN


def _prep_body(w1r_ref, b1r_ref, att_ref, w2r_ref, vt_ref, gt_ref, cb_ref, m_ref):
    zeros4 = jnp.zeros((H_HEADS, C_IN), dtype=jnp.float32)
    vrows = [jnp.dot(att_ref[h : h + 1, :], w1r_ref[h],
                     preferred_element_type=jnp.float32)
             for h in range(H_HEADS)]
    vt_ref[...] = lax.concatenate(vrows + [zeros4], 0)
    grows = [lax.dot_general(b1r_ref[h : h + 1, :], w2r_ref[h],
                             (((1,), (1,)), ((), ())),
                             preferred_element_type=jnp.float32)
             for h in range(H_HEADS)]
    gt_ref[...] = lax.concatenate(grows + [zeros4], 0)
    cbv = jnp.sum(att_ref[...] * b1r_ref[...], axis=1, keepdims=True)
    cb8 = lax.concatenate([cbv, jnp.zeros((H_HEADS, 1), jnp.float32)], 0)
    cb_ref[...] = jnp.broadcast_to(cb8, (HP, 128))
    for h in range(H_HEADS):
        m_ref[h] = jnp.dot(w2r_ref[h], w1r_ref[h],
                           preferred_element_type=jnp.float32)


def _scatter_body(x_ref, idx_ref, vt_ref, cb_ref, yout_ref, sout_ref,
                  xbf_ref, ys_ref, ss_ref):
    j = pl.program_id(1)

    @pl.when(j == 0)
    def _init():
        ys_ref[...] = jnp.zeros_like(ys_ref)
        ss_ref[...] = jnp.zeros_like(ss_ref)

    xb = x_ref[0].astype(jnp.bfloat16)
    xbf_ref[0] = xb
    idxv = idx_ref[0, 0]
    vtb = vt_ref[...].astype(jnp.bfloat16)
    logits = jnp.dot(vtb, xb, preferred_element_type=jnp.float32)
    logits = logits + cb_ref[:, :1]
    ex = jnp.exp(logits)
    exb = ex.astype(jnp.bfloat16)
    oht = (lax.broadcasted_iota(jnp.int32, (EP, BN), 0) == idxv
           ).astype(jnp.bfloat16)
    ss_ref[...] += lax.dot_general(exb, oht, (((1,), (1,)), ((), ())),
                                   preferred_element_type=jnp.float32)
    for h in range(H_HEADS):
        ys_ref[h] += lax.dot_general(xb * exb[h : h + 1, :], oht,
                                     (((1,), (1,)), ((), ())),
                                     preferred_element_type=jnp.float32)

    @pl.when(j == NB - 1)
    def _flush():
        yout_ref[0] = ys_ref[...]
        sout_ref[0] = ss_ref[...]


def _edge_body(ys_ref, ss_ref, m_ref, gt_ref, b2_ref, et_ref):
    s = ss_ref[0]
    cnt = jnp.maximum(s[H_HEADS : H_HEADS + 1, :], 1.0)
    cinv = 1.0 / cnt
    dn = jnp.sum(s, axis=1, keepdims=True)
    sn = s * (cinv / dn)
    acc = lax.dot_general(gt_ref[...], sn, (((0,), (0,)), ((), ())),
                          preferred_element_type=jnp.float32)
    for h in range(H_HEADS):
        dnh = jnp.sum(s[h : h + 1, :], axis=1, keepdims=True)
        zh = ys_ref[0, h] * (cinv / dnh)
        acc = acc + jnp.dot(m_ref[h], zh, preferred_element_type=jnp.float32)
    et_ref[0] = acc + b2_ref[...]


def _out_body(x_ref, idx_ref, et_ref, wr_ref, br_ref, o_ref):
    idxv = idx_ref[0, 0]
    oht = (lax.broadcasted_iota(jnp.int32, (EP, BN), 0) == idxv
           ).astype(jnp.bfloat16)
    etb = et_ref[0].astype(jnp.bfloat16)
    g = jnp.dot(etb, oht, preferred_element_type=jnp.float32)
    wrb = wr_ref[...].astype(jnp.bfloat16)
    r = jnp.dot(wrb, x_ref[0], preferred_element_type=jnp.float32)
    v = g + r + br_ref[...]
    o_ref[0] = jnp.where(v > 0, v, jnp.exp(jnp.minimum(v, 0.0)) - 1.0)


def kernel(x, hyperedge_labels, W1, b1, W2, b2, attention, Wr, br):
    B, C, H, W = x.shape
    N = H * W
    x3 = x.reshape(B, C, N)
    idx = hyperedge_labels.astype(jnp.int32).reshape(B, NB, 1, BN)

    w1r = W1.reshape(H_HEADS, C_OUT, C)
    b1r = b1.reshape(H_HEADS, C_OUT)
    att = attention.reshape(H_HEADS, C_OUT)
    w2r = jnp.transpose(W2.reshape(C_OUT, H_HEADS, C_OUT), (1, 0, 2))
    b2c = b2.reshape(C_OUT, 1)
    brc = br.reshape(C_OUT, 1)

    vt, gt, cb, m = pl.pallas_call(
        _prep_body,
        out_shape=[
            jax.ShapeDtypeStruct((HP, C_IN), jnp.float32),
            jax.ShapeDtypeStruct((HP, C_IN), jnp.float32),
            jax.ShapeDtypeStruct((HP, 128), jnp.float32),
            jax.ShapeDtypeStruct((H_HEADS, C_OUT, C_IN), jnp.float32),
        ],
    )(w1r, b1r, att, w2r)

    yacc, sacc, xbf = pl.pallas_call(
        _scatter_body,
        grid=(B, NB),
        in_specs=[
            pl.BlockSpec((1, C, BN), lambda b, j: (b, 0, j)),
            pl.BlockSpec((1, 1, 1, BN), lambda b, j: (b, j, 0, 0)),
            pl.BlockSpec((HP, C_IN), lambda b, j: (0, 0)),
            pl.BlockSpec((HP, 128), lambda b, j: (0, 0)),
        ],
        out_specs=[
            pl.BlockSpec((1, H_HEADS, C_IN, EP), lambda b, j: (b, 0, 0, 0)),
            pl.BlockSpec((1, HP, EP), lambda b, j: (b, 0, 0)),
            pl.BlockSpec((1, C, BN), lambda b, j: (b, 0, j)),
        ],
        out_shape=[
            jax.ShapeDtypeStruct((B, H_HEADS, C_IN, EP), jnp.float32),
            jax.ShapeDtypeStruct((B, HP, EP), jnp.float32),
            jax.ShapeDtypeStruct((B, C, N), jnp.bfloat16),
        ],
        scratch_shapes=[
            pltpu.VMEM((H_HEADS, C_IN, EP), jnp.float32),
            pltpu.VMEM((HP, EP), jnp.float32),
        ],
    )(x3, idx, vt, cb)

    et = pl.pallas_call(
        _edge_body,
        grid=(B,),
        in_specs=[
            pl.BlockSpec((1, H_HEADS, C_IN, EP), lambda b: (b, 0, 0, 0)),
            pl.BlockSpec((1, HP, EP), lambda b: (b, 0, 0)),
            pl.BlockSpec((H_HEADS, C_OUT, C_IN), lambda b: (0, 0, 0)),
            pl.BlockSpec((HP, C_IN), lambda b: (0, 0)),
            pl.BlockSpec((C_OUT, 1), lambda b: (0, 0)),
        ],
        out_specs=pl.BlockSpec((1, C_OUT, EP), lambda b: (b, 0, 0)),
        out_shape=jax.ShapeDtypeStruct((B, C_OUT, EP), jnp.float32),
    )(yacc, sacc, m, gt, b2c)

    out = pl.pallas_call(
        _out_body,
        grid=(B, NB),
        in_specs=[
            pl.BlockSpec((1, C, BN), lambda b, j: (b, 0, j)),
            pl.BlockSpec((1, 1, 1, BN), lambda b, j: (b, j, 0, 0)),
            pl.BlockSpec((1, C_OUT, EP), lambda b, j: (b, 0, 0)),
            pl.BlockSpec((C_OUT, C_IN), lambda b, j: (0, 0)),
            pl.BlockSpec((C_OUT, 1), lambda b, j: (0, 0)),
        ],
        out_specs=pl.BlockSpec((1, C_OUT, BN), lambda b, j: (b, 0, j)),
        out_shape=jax.ShapeDtypeStruct((B, C_OUT, N), jnp.float32),
    )(xbf, idx, et, Wr, brc)

    return out.reshape(B, C_OUT, H, W)


# speedup vs baseline: 26.3800x; 1.0374x over previous
import jax
import jax.numpy as jnp
from jax import lax
from jax.experimental import pallas as pl
from jax.experimental.pallas import tpu as pltpu

H_HEADS = 4
HP = 8
C_IN = 192
C_OUT = 192
E_EDGES = 200
EP = 256
BN = 3584
N_TOT = 224 * 224
NB = N_TOT // BN


def _prep_body(w1r_ref, b1r_ref, att_ref, w2r_ref, vt_ref, gt_ref, cb_ref, m_ref):
    zeros4 = jnp.zeros((H_HEADS, C_IN), dtype=jnp.float32)
    vrows = [jnp.dot(att_ref[h : h + 1, :], w1r_ref[h],
                     preferred_element_type=jnp.float32)
             for h in range(H_HEADS)]
    vt_ref[...] = lax.concatenate(vrows + [zeros4], 0)
    grows = [lax.dot_general(b1r_ref[h : h + 1, :], w2r_ref[h],
                             (((1,), (1,)), ((), ())),
                             preferred_element_type=jnp.float32)
             for h in range(H_HEADS)]
    gt_ref[...] = lax.concatenate(grows + [zeros4], 0)
    cbv = jnp.sum(att_ref[...] * b1r_ref[...], axis=1, keepdims=True)
    cb8 = lax.concatenate([cbv, jnp.zeros((H_HEADS, 1), jnp.float32)], 0)
    cb_ref[...] = jnp.broadcast_to(cb8, (HP, 128))
    for h in range(H_HEADS):
        m_ref[h] = jnp.dot(w2r_ref[h], w1r_ref[h],
                           preferred_element_type=jnp.float32)


def _fused_body(x_ref, idx_ref, vt_ref, cb_ref, m_ref, gt_ref, b2_ref,
                wr_ref, br_ref, o_ref, xbf_ref, ys_ref, ss_ref, et_ref):
    p = pl.program_id(1)
    j = pl.program_id(2)
    idxv = idx_ref[0, 0]
    oht = (lax.broadcasted_iota(jnp.int32, (EP, BN), 0) == idxv
           ).astype(jnp.bfloat16)

    @pl.when(p == 0)
    def _scatter():
        @pl.when(j == 0)
        def _init():
            ys_ref[...] = jnp.zeros_like(ys_ref)
            ss_ref[...] = jnp.zeros_like(ss_ref)

        xb = x_ref[0].astype(jnp.bfloat16)
        xbf_ref[j] = xb
        vtb = vt_ref[...].astype(jnp.bfloat16)
        logits = jnp.dot(vtb, xb, preferred_element_type=jnp.float32)
        logits = logits + cb_ref[:, :1]
        ex = jnp.exp(logits)
        exb = ex.astype(jnp.bfloat16)
        ss_ref[...] += lax.dot_general(exb, oht, (((1,), (1,)), ((), ())),
                                       preferred_element_type=jnp.float32)
        for h in range(H_HEADS):
            ys_ref[h] += lax.dot_general(xb * exb[h : h + 1, :], oht,
                                         (((1,), (1,)), ((), ())),
                                         preferred_element_type=jnp.float32)

    @pl.when((p == 1) & (j == 0))
    def _edge():
        s = ss_ref[...]
        cnt = jnp.maximum(s[H_HEADS : H_HEADS + 1, :], 1.0)
        cinv = 1.0 / cnt
        dn = jnp.sum(s, axis=1, keepdims=True)
        sn = s * (cinv / dn)
        acc = lax.dot_general(gt_ref[...], sn, (((0,), (0,)), ((), ())),
                              preferred_element_type=jnp.float32)
        for h in range(H_HEADS):
            dnh = jnp.sum(s[h : h + 1, :], axis=1, keepdims=True)
            zh = ys_ref[h] * (cinv / dnh)
            acc = acc + jnp.dot(m_ref[h], zh,
                                preferred_element_type=jnp.float32)
        et_ref[...] = acc + b2_ref[...]

    @pl.when(p == 1)
    def _output():
        etb = et_ref[...].astype(jnp.bfloat16)
        g = jnp.dot(etb, oht, preferred_element_type=jnp.float32)
        wrb = wr_ref[...].astype(jnp.bfloat16)
        r = jnp.dot(wrb, xbf_ref[j], preferred_element_type=jnp.float32)
        v = g + r + br_ref[...]
        o_ref[0] = jnp.where(v > 0, v, jnp.exp(jnp.minimum(v, 0.0)) - 1.0)


def kernel(x, hyperedge_labels, W1, b1, W2, b2, attention, Wr, br):
    B, C, H, W = x.shape
    N = H * W
    x3 = x.reshape(B, C, N)
    idx = hyperedge_labels.astype(jnp.int32).reshape(B, NB, 1, BN)

    w1r = W1.reshape(H_HEADS, C_OUT, C)
    b1r = b1.reshape(H_HEADS, C_OUT)
    att = attention.reshape(H_HEADS, C_OUT)
    w2r = jnp.transpose(W2.reshape(C_OUT, H_HEADS, C_OUT), (1, 0, 2))
    b2c = b2.reshape(C_OUT, 1)
    brc = br.reshape(C_OUT, 1)

    vt, gt, cb, m = pl.pallas_call(
        _prep_body,
        out_shape=[
            jax.ShapeDtypeStruct((HP, C_IN), jnp.float32),
            jax.ShapeDtypeStruct((HP, C_IN), jnp.float32),
            jax.ShapeDtypeStruct((HP, 128), jnp.float32),
            jax.ShapeDtypeStruct((H_HEADS, C_OUT, C_IN), jnp.float32),
        ],
    )(w1r, b1r, att, w2r)

    out = pl.pallas_call(
        _fused_body,
        grid=(B, 2, NB),
        in_specs=[
            pl.BlockSpec((1, C, BN),
                         lambda b, p, j: (b, 0, j * (1 - p) + (NB - 1) * p)),
            pl.BlockSpec((1, 1, 1, BN), lambda b, p, j: (b, j, 0, 0)),
            pl.BlockSpec((HP, C_IN), lambda b, p, j: (0, 0)),
            pl.BlockSpec((HP, 128), lambda b, p, j: (0, 0)),
            pl.BlockSpec((H_HEADS, C_OUT, C_IN), lambda b, p, j: (0, 0, 0)),
            pl.BlockSpec((HP, C_IN), lambda b, p, j: (0, 0)),
            pl.BlockSpec((C_OUT, 1), lambda b, p, j: (0, 0)),
            pl.BlockSpec((C_OUT, C_IN), lambda b, p, j: (0, 0)),
            pl.BlockSpec((C_OUT, 1), lambda b, p, j: (0, 0)),
        ],
        out_specs=pl.BlockSpec((1, C_OUT, BN), lambda b, p, j: (b, 0, j)),
        out_shape=jax.ShapeDtypeStruct((B, C_OUT, N), jnp.float32),
        scratch_shapes=[
            pltpu.VMEM((NB, C_IN, BN), jnp.bfloat16),
            pltpu.VMEM((H_HEADS, C_IN, EP), jnp.float32),
            pltpu.VMEM((HP, EP), jnp.float32),
            pltpu.VMEM((C_OUT, EP), jnp.float32),
        ],
    )(x3, idx, vt, cb, m, gt, b2c, Wr, brc)

    return out.reshape(B, C_OUT, H, W)
